# baseline (speedup 1.0000x reference)
import functools

import jax
import jax.numpy as jnp
from jax import lax
from jax.experimental import pallas as pl
from jax.experimental.pallas import tpu as pltpu
from jax.experimental.pallas import tpu_sc as plsc

N_NODES = 10000
HIDDEN = 128
N_EDGES = 320000
DROPOUT_RATE = 0.25

NC = 2
NS = 16
NW = NC * NS
CHUNK = 128
EPW_CHUNKS = 80
EPW = EPW_CHUNKS * CHUNK
E_PAD = NW * EPW
ACC_ROWS = 10112
ZROWS = 16
ROWS_PER_SUB = ACC_ROWS // NS


def _sc_scatter(x, src, dst):
    mesh = plsc.VectorSubcoreMesh(core_axis_name="c", subcore_axis_name="s")

    @functools.partial(
        pl.kernel,
        mesh=mesh,
        out_type=jax.ShapeDtypeStruct((NC, ACC_ROWS, HIDDEN), jnp.float32),
        scratch_types=[
            pltpu.VMEM((EPW_CHUNKS + 1, CHUNK), jnp.int32),
            pltpu.VMEM((CHUNK,), jnp.int32),
            pltpu.VMEM((CHUNK,), jnp.int32),
            pltpu.VMEM((CHUNK, HIDDEN), jnp.float32),
            pltpu.VMEM((CHUNK, HIDDEN), jnp.float32),
            pltpu.VMEM((ZROWS, HIDDEN), jnp.float32),
            pltpu.VMEM_SHARED((ACC_ROWS, HIDDEN), jnp.float32),
            pltpu.SemaphoreType.DMA,
            pltpu.SemaphoreType.DMA,
        ],
    )
    def k(x_hbm, src_hbm, dst_hbm, out_hbm, srcs_v, dst_a, dst_b, rows_a,
          rows_b, z_v, acc_sh, sem, dsem):
        c = lax.axis_index("c")
        s = lax.axis_index("s")
        wid = c * NS + s

        pltpu.sync_copy(src_hbm.at[wid], srcs_v)

        def zrow(i, carry):
            for j in range(HIDDEN // 16):
                z_v[i, pl.ds(j * 16, 16)] = jnp.zeros((16,), jnp.float32)
            return carry

        lax.fori_loop(0, ZROWS, zrow, 0)

        def zacc(i, carry):
            pltpu.sync_copy(z_v, acc_sh.at[pl.ds(s * ROWS_PER_SUB + i * ZROWS, ZROWS)])
            return carry

        lax.fori_loop(0, ROWS_PER_SUB // ZROWS, zacc, 0)
        plsc.subcore_barrier()

        def gather_start(j, buf):
            pltpu.async_copy(x_hbm.at[srcs_v.at[j]], buf, sem)

        def gather_wait(buf):
            pltpu.make_async_copy(x_hbm.at[pl.ds(0, CHUNK)], buf, sem).wait()

        def dst_start(j, buf):
            pltpu.async_copy(dst_hbm.at[wid, j], buf, dsem)

        def dst_wait(buf):
            pltpu.make_async_copy(dst_hbm.at[0, 0], buf, dsem).wait()

        gather_start(0, rows_a)
        dst_start(0, dst_a)

        def body(jj, carry):
            j0 = jj * 2
            gather_wait(rows_a)
            dst_wait(dst_a)
            gather_start(j0 + 1, rows_b)
            dst_start(j0 + 1, dst_b)
            pltpu.sync_copy(rows_a, acc_sh.at[dst_a], add=True)
            gather_wait(rows_b)
            dst_wait(dst_b)
            gather_start(j0 + 2, rows_a)
            dst_start(j0 + 2, dst_a)
            pltpu.sync_copy(rows_b, acc_sh.at[dst_b], add=True)
            return carry

        lax.fori_loop(0, EPW_CHUNKS // 2, body, 0)
        gather_wait(rows_a)
        dst_wait(dst_a)
        plsc.subcore_barrier()

        pltpu.sync_copy(
            acc_sh.at[pl.ds(s * ROWS_PER_SUB, ROWS_PER_SUB)],
            out_hbm.at[c, pl.ds(s * ROWS_PER_SUB, ROWS_PER_SUB)],
        )

    return k(x, src, dst)


def _tc_mlp(x, p0, p1, mask, W1, b1, W2, b2):
    BLK = 1000

    def body(x_ref, p0_ref, p1_ref, m_ref, w1_ref, b1_ref, w2_ref, b2_ref, o_ref):
        h = (x_ref[...] + p0_ref[...] + p1_ref[...]) * m_ref[...]
        h = jnp.dot(h, w1_ref[...], preferred_element_type=jnp.float32) + b1_ref[...]
        h = jnp.maximum(h, 0.0)
        o_ref[...] = jnp.dot(h, w2_ref[...], preferred_element_type=jnp.float32) + b2_ref[...]

    row_spec = pl.BlockSpec((BLK, HIDDEN), lambda i: (i, 0))
    full_spec = pl.BlockSpec((HIDDEN, HIDDEN), lambda i: (0, 0))
    vec_spec = pl.BlockSpec((1, HIDDEN), lambda i: (0, 0))
    return pl.pallas_call(
        body,
        grid=(N_NODES // BLK,),
        in_specs=[row_spec, row_spec, row_spec, vec_spec, full_spec, vec_spec,
                  full_spec, vec_spec],
        out_specs=row_spec,
        out_shape=jax.ShapeDtypeStruct((N_NODES, HIDDEN), jnp.float32),
    )(x, p0, p1, mask, W1, b1, W2, b2)


def kernel(x, edge_index, W1, b1, W2, b2):
    mask = jax.random.bernoulli(
        jax.random.key(42), p=1.0 - DROPOUT_RATE, shape=(HIDDEN,)
    ).astype(x.dtype)
    src = edge_index[0].astype(jnp.int32)
    dst = edge_index[1].astype(jnp.int32)
    pad = E_PAD - N_EDGES
    src = jnp.concatenate([src, jnp.zeros((pad,), jnp.int32)])
    dst = jnp.concatenate([dst, jnp.full((pad,), N_NODES, jnp.int32)])
    src = src.reshape(NW, EPW_CHUNKS, CHUNK)
    src = jnp.concatenate([src, jnp.zeros((NW, 1, CHUNK), jnp.int32)], axis=1)
    dst = dst.reshape(NW, EPW_CHUNKS, CHUNK)
    dst = jnp.concatenate([dst, jnp.full((NW, 1, CHUNK), N_NODES, jnp.int32)], axis=1)
    partials = _sc_scatter(x, src, dst)
    return _tc_mlp(
        x, partials[0, :N_NODES], partials[1, :N_NODES], mask.reshape(1, HIDDEN),
        W1, b1.reshape(1, HIDDEN), W2, b2.reshape(1, HIDDEN),
    )


# speedup vs baseline: 1.1541x; 1.1541x over previous
import functools

import jax
import jax.numpy as jnp
from jax import lax
from jax.experimental import pallas as pl
from jax.experimental.pallas import tpu as pltpu
from jax.experimental.pallas import tpu_sc as plsc

N_NODES = 10000
HIDDEN = 128
N_EDGES = 320000
DROPOUT_RATE = 0.25

NC = 2
NS = 16
NW = NC * NS
CHUNK = 64
NB = 4
CPW = 160
NBODY = CPW // NB
EPW = CPW * CHUNK
E_PAD = NW * EPW
ACC_ROWS = 10112
ZROWS = 16
ROWS_PER_SUB = ACC_ROWS // NS


def _sc_scatter(x, src, dst):
    mesh = plsc.VectorSubcoreMesh(core_axis_name="c", subcore_axis_name="s")

    @functools.partial(
        pl.kernel,
        mesh=mesh,
        out_type=jax.ShapeDtypeStruct((NC, ACC_ROWS, HIDDEN), jnp.float32),
        scratch_types=[
            pltpu.VMEM((NB, CHUNK), jnp.int32),
            pltpu.VMEM((NB, CHUNK), jnp.int32),
        ] + [
            pltpu.VMEM((CHUNK, HIDDEN), jnp.float32) for _ in range(NB)
        ] + [
            pltpu.VMEM((ZROWS, HIDDEN), jnp.float32),
            pltpu.VMEM_SHARED((ACC_ROWS, HIDDEN), jnp.float32),
        ] + [pltpu.SemaphoreType.DMA for _ in range(NB)],
    )
    def k(x_hbm, src_hbm, dst_hbm, out_hbm, src_i, dst_i, *rest):
        rows = rest[:NB]
        z_v = rest[NB]
        acc_sh = rest[NB + 1]
        sems = rest[NB + 2:]
        c = lax.axis_index("c")
        s = lax.axis_index("s")
        wid = c * NS + s

        def zrow(i, carry):
            for j in range(HIDDEN // 16):
                z_v[i, pl.ds(j * 16, 16)] = jnp.zeros((16,), jnp.float32)
            return carry

        lax.fori_loop(0, ZROWS, zrow, 0)

        def zacc(i, carry):
            pltpu.sync_copy(z_v, acc_sh.at[pl.ds(s * ROWS_PER_SUB + i * ZROWS, ZROWS)])
            return carry

        lax.fori_loop(0, ROWS_PER_SUB // ZROWS, zacc, 0)
        plsc.subcore_barrier()

        def body(jj, carry):
            pltpu.sync_copy(src_hbm.at[wid, pl.ds(jj * NB, NB)], src_i)
            pltpu.sync_copy(dst_hbm.at[wid, pl.ds(jj * NB, NB)], dst_i)
            copies = [
                pltpu.async_copy(x_hbm.at[src_i.at[b]], rows[b], sems[b])
                for b in range(NB)
            ]
            for b in range(NB):
                copies[b].wait()
                pltpu.sync_copy(rows[b], acc_sh.at[dst_i.at[b]], add=True)
            return carry

        lax.fori_loop(0, NBODY, body, 0)
        plsc.subcore_barrier()

        pltpu.sync_copy(
            acc_sh.at[pl.ds(s * ROWS_PER_SUB, ROWS_PER_SUB)],
            out_hbm.at[c, pl.ds(s * ROWS_PER_SUB, ROWS_PER_SUB)],
        )

    return k(x, src, dst)


def _tc_mlp(x, p0, p1, mask, W1, b1, W2, b2):
    BLK = 1000

    def body(x_ref, p0_ref, p1_ref, m_ref, w1_ref, b1_ref, w2_ref, b2_ref, o_ref):
        h = (x_ref[...] + p0_ref[...] + p1_ref[...]) * m_ref[...]
        h = jnp.dot(h, w1_ref[...], preferred_element_type=jnp.float32) + b1_ref[...]
        h = jnp.maximum(h, 0.0)
        o_ref[...] = jnp.dot(h, w2_ref[...], preferred_element_type=jnp.float32) + b2_ref[...]

    row_spec = pl.BlockSpec((BLK, HIDDEN), lambda i: (i, 0))
    full_spec = pl.BlockSpec((HIDDEN, HIDDEN), lambda i: (0, 0))
    vec_spec = pl.BlockSpec((1, HIDDEN), lambda i: (0, 0))
    return pl.pallas_call(
        body,
        grid=(N_NODES // BLK,),
        in_specs=[row_spec, row_spec, row_spec, vec_spec, full_spec, vec_spec,
                  full_spec, vec_spec],
        out_specs=row_spec,
        out_shape=jax.ShapeDtypeStruct((N_NODES, HIDDEN), jnp.float32),
    )(x, p0, p1, mask, W1, b1, W2, b2)


def kernel(x, edge_index, W1, b1, W2, b2):
    mask = jax.random.bernoulli(
        jax.random.key(42), p=1.0 - DROPOUT_RATE, shape=(HIDDEN,)
    ).astype(x.dtype)
    src = edge_index[0].astype(jnp.int32)
    dst = edge_index[1].astype(jnp.int32)
    pad = E_PAD - N_EDGES
    src = jnp.concatenate([src, jnp.zeros((pad,), jnp.int32)])
    dst = jnp.concatenate([dst, jnp.full((pad,), N_NODES, jnp.int32)])
    src = src.reshape(NW, CPW, CHUNK)
    dst = dst.reshape(NW, CPW, CHUNK)
    partials = _sc_scatter(x, src, dst)
    return _tc_mlp(
        x, partials[0, :N_NODES], partials[1, :N_NODES], mask.reshape(1, HIDDEN),
        W1, b1.reshape(1, HIDDEN), W2, b2.reshape(1, HIDDEN),
    )


# speedup vs baseline: 1.2787x; 1.1079x over previous
import functools

import jax
import jax.numpy as jnp
from jax import lax
from jax.experimental import pallas as pl
from jax.experimental.pallas import tpu as pltpu
from jax.experimental.pallas import tpu_sc as plsc

N_NODES = 10000
HIDDEN = 128
N_EDGES = 320000
DROPOUT_RATE = 0.25

NC = 2
NS = 16
NW = NC * NS
CHUNK = 128
CPW = 80
KB = 16
NBODY = CPW // KB
EPW = CPW * CHUNK
E_PAD = NW * EPW
ACC_ROWS = 10112
ROWS_PER_SUB = ACC_ROWS // NS


def _sc_scatter(x, src, dst, zeros_rows):
    mesh = plsc.VectorSubcoreMesh(core_axis_name="c", subcore_axis_name="s")

    @functools.partial(
        pl.kernel,
        mesh=mesh,
        out_type=jax.ShapeDtypeStruct((NC, ACC_ROWS, HIDDEN), jnp.float32),
        scratch_types=[
            pltpu.VMEM((KB, CHUNK), jnp.int32),
            pltpu.VMEM((CHUNK,), jnp.int32),
            pltpu.VMEM((CHUNK,), jnp.int32),
            pltpu.VMEM((CHUNK, HIDDEN), jnp.float32),
            pltpu.VMEM((CHUNK, HIDDEN), jnp.float32),
            pltpu.VMEM_SHARED((ACC_ROWS, HIDDEN), jnp.float32),
            pltpu.SemaphoreType.DMA,
            pltpu.SemaphoreType.DMA,
            pltpu.SemaphoreType.DMA,
            pltpu.SemaphoreType.DMA,
            pltpu.SemaphoreType.DMA,
            pltpu.SemaphoreType.DMA,
        ],
    )
    def k(x_hbm, src_hbm, dst_hbm, zero_hbm, out_hbm, src_i, dba, dbb,
          rows_a, rows_b, acc_sh, gs_a, gs_b, ss_a, ss_b, ds_a, ds_b):
        c = lax.axis_index("c")
        s = lax.axis_index("s")
        wid = c * NS + s

        pltpu.sync_copy(zero_hbm, acc_sh.at[pl.ds(s * ROWS_PER_SUB, ROWS_PER_SUB)])
        plsc.subcore_barrier()

        rows = (rows_a, rows_b)
        dbuf = (dba, dbb)
        gsem = (gs_a, gs_b)
        ssem = (ss_a, ss_b)
        dsem = (ds_a, ds_b)

        def body(p, carry):
            pltpu.sync_copy(src_hbm.at[wid, pl.ds(p * KB, KB)], src_i)
            scat = [None] * KB
            for t in range(KB):
                b = t % 2
                if t >= 2:
                    scat[t - 2].wait()
                dcp = pltpu.async_copy(dst_hbm.at[wid, p * KB + t], dbuf[b], dsem[b])
                gcp = pltpu.async_copy(x_hbm.at[src_i.at[t]], rows[b], gsem[b])
                gcp.wait()
                dcp.wait()
                scat[t] = pltpu.async_copy(rows[b], acc_sh.at[dbuf[b]], ssem[b], add=True)
            scat[KB - 2].wait()
            scat[KB - 1].wait()
            return carry

        lax.fori_loop(0, NBODY, body, 0)
        plsc.subcore_barrier()

        pltpu.sync_copy(
            acc_sh.at[pl.ds(s * ROWS_PER_SUB, ROWS_PER_SUB)],
            out_hbm.at[c, pl.ds(s * ROWS_PER_SUB, ROWS_PER_SUB)],
        )

    return k(x, src, dst, zeros_rows)


def _tc_mlp(x, p0, p1, mask, W1, b1, W2, b2):
    BLK = 1000

    def body(x_ref, p0_ref, p1_ref, m_ref, w1_ref, b1_ref, w2_ref, b2_ref, o_ref):
        h = (x_ref[...] + p0_ref[...] + p1_ref[...]) * m_ref[...]
        h = jnp.dot(h, w1_ref[...], preferred_element_type=jnp.float32) + b1_ref[...]
        h = jnp.maximum(h, 0.0)
        o_ref[...] = jnp.dot(h, w2_ref[...], preferred_element_type=jnp.float32) + b2_ref[...]

    row_spec = pl.BlockSpec((BLK, HIDDEN), lambda i: (i, 0))
    full_spec = pl.BlockSpec((HIDDEN, HIDDEN), lambda i: (0, 0))
    vec_spec = pl.BlockSpec((1, HIDDEN), lambda i: (0, 0))
    return pl.pallas_call(
        body,
        grid=(N_NODES // BLK,),
        in_specs=[row_spec, row_spec, row_spec, vec_spec, full_spec, vec_spec,
                  full_spec, vec_spec],
        out_specs=row_spec,
        out_shape=jax.ShapeDtypeStruct((N_NODES, HIDDEN), jnp.float32),
    )(x, p0, p1, mask, W1, b1, W2, b2)


def kernel(x, edge_index, W1, b1, W2, b2):
    mask = jax.random.bernoulli(
        jax.random.key(42), p=1.0 - DROPOUT_RATE, shape=(HIDDEN,)
    ).astype(x.dtype)
    src = edge_index[0].astype(jnp.int32)
    dst = edge_index[1].astype(jnp.int32)
    pad = E_PAD - N_EDGES
    src = jnp.concatenate([src, jnp.zeros((pad,), jnp.int32)])
    dst = jnp.concatenate([dst, jnp.full((pad,), N_NODES, jnp.int32)])
    src = src.reshape(NW, CPW, CHUNK)
    dst = dst.reshape(NW, CPW, CHUNK)
    zeros_rows = jnp.zeros((ROWS_PER_SUB, HIDDEN), jnp.float32)
    partials = _sc_scatter(x, src, dst, zeros_rows)
    return _tc_mlp(
        x, partials[0, :N_NODES], partials[1, :N_NODES], mask.reshape(1, HIDDEN),
        W1, b1.reshape(1, HIDDEN), W2, b2.reshape(1, HIDDEN),
    )
